# Initial kernel scaffold; baseline (speedup 1.0000x reference)
import jax
import jax.numpy as jnp
from jax.experimental import pallas as pl


def kernel(x, edge_index, batch, W1, b1, W2, b2, W3, b3, W4, b4, W5, b5):
    raise NotImplementedError("write your pallas kernel here")


# speedup vs baseline: 10.9137x; 10.9137x over previous
import functools

import jax
import jax.numpy as jnp
from jax import lax
from jax.experimental import pallas as pl
from jax.experimental.pallas import tpu as pltpu
from jax.experimental.pallas import tpu_sc as plsc

N = 10000
NUM_GRAPHS = 64
N_PAD = 10240
E_PAD = 327680
GROUPS = E_PAD // 128
GROUPS_PER_TILE = GROUPS // 32
ROWS_PER_SUBCORE = N_PAD // 16

_f32 = jnp.float32


@functools.lru_cache(maxsize=None)
def _make_sc_scatter(d: int, ch: int):
    per_tile = E_PAD // 32
    assert per_tile % ch == 0
    steps = per_tile // ch
    mesh = plsc.VectorSubcoreMesh(core_axis_name="c", subcore_axis_name="s")

    def body(y_hbm, src_hbm, dst_hbm, zero_hbm, u_hbm,
             src_v, dst_v, rows_v, u_sh, sem):
        c = lax.axis_index("c")
        s = lax.axis_index("s")
        r0 = s * ROWS_PER_SUBCORE
        pltpu.sync_copy(zero_hbm.at[pl.ds(r0, ROWS_PER_SUBCORE)],
                        u_sh.at[pl.ds(r0, ROWS_PER_SUBCORE)])
        plsc.subcore_barrier()
        base = (c * 16 + s) * per_tile

        def step(j, carry):
            e0 = base + j * ch
            pltpu.sync_copy(src_hbm.at[pl.ds(e0, ch)], src_v)
            pltpu.sync_copy(dst_hbm.at[pl.ds(e0, ch)], dst_v)
            pltpu.async_copy(y_hbm.at[src_v], rows_v, sem).wait()
            pltpu.sync_copy(rows_v, u_sh.at[dst_v], add=True)
            return carry

        lax.fori_loop(0, steps, step, 0)
        plsc.subcore_barrier()
        pltpu.sync_copy(u_sh.at[pl.ds(r0, ROWS_PER_SUBCORE)],
                        u_hbm.at[c, pl.ds(r0, ROWS_PER_SUBCORE)])

    return pl.kernel(
        body,
        out_type=jax.ShapeDtypeStruct((2, N_PAD, d), _f32),
        mesh=mesh,
        scratch_types=[
            pltpu.VMEM((ch,), jnp.int32),
            pltpu.VMEM((ch,), jnp.int32),
            pltpu.VMEM((ch, d), _f32),
            pltpu.VMEM_SHARED((N_PAD, d), _f32),
            pltpu.SemaphoreType.DMA,
        ],
        compiler_params=pltpu.CompilerParams(use_tc_tiling_on_sc=False),
    )


def _sc_scatter(y, src1d, dst1d, d, ch):
    zero = jnp.zeros((N_PAD, d), _f32)
    return _make_sc_scatter(d, ch)(y, src1d, dst1d, zero)


def _rowmask():
    rows = lax.broadcasted_iota(jnp.int32, (N_PAD, 1), 0)
    return (rows < N).astype(_f32)


def _tc_b1(x_p, w1p, u_deg):
    def body(x_ref, w_ref, ud_ref, y_ref, dinv_ref):
        deg = ud_ref[0, :, :1] + ud_ref[1, :, :1] + 1.0
        dinv = lax.rsqrt(deg)
        t = jnp.dot(x_ref[...], w_ref[...], preferred_element_type=_f32)
        y_ref[...] = _rowmask() * (dinv * t)
        dinv_ref[...] = dinv

    return pl.pallas_call(
        body,
        out_shape=(jax.ShapeDtypeStruct((N_PAD, 16), _f32),
                   jax.ShapeDtypeStruct((N_PAD, 1), _f32)),
    )(x_p, w1p, u_deg)


def _tc_mid(u, y, dinv, w, b, dout):
    def body(u_ref, y_ref, dinv_ref, w_ref, b_ref, out_ref):
        dinv = dinv_ref[...]
        g = dinv * (u_ref[0] + u_ref[1] + y_ref[...])
        h = jnp.maximum(jnp.dot(g, w_ref[...], preferred_element_type=_f32)
                        + b_ref[...], 0.0)
        out_ref[...] = _rowmask() * (dinv * h)

    return pl.pallas_call(
        body,
        out_shape=jax.ShapeDtypeStruct((N_PAD, dout), _f32),
    )(u, y, dinv, w, b)


def _tc_final(u, y, dinv, w, b, batch_p):
    def body(u_ref, y_ref, dinv_ref, w_ref, b_ref, bat_ref, out_ref):
        g = dinv_ref[...] * (u_ref[0] + u_ref[1] + y_ref[...])
        h = jnp.maximum(jnp.dot(g, w_ref[...], preferred_element_type=_f32)
                        + b_ref[...], 0.0)
        gid = lax.broadcasted_iota(jnp.int32, (N_PAD, NUM_GRAPHS), 1)
        rows = lax.broadcasted_iota(jnp.int32, (N_PAD, NUM_GRAPHS), 0)
        m = ((bat_ref[...] == gid) & (rows < N)).astype(_f32)
        sums = lax.dot_general(m, h, (((0,), (0,)), ((), ())),
                               preferred_element_type=_f32)
        counts = jnp.sum(m, axis=0)[:, None]
        out_ref[...] = sums / jnp.maximum(counts, 1.0)

    return pl.pallas_call(
        body,
        out_shape=jax.ShapeDtypeStruct((NUM_GRAPHS, 256), _f32),
    )(u, y, dinv, w, b, batch_p)


def kernel(x, edge_index, batch, W1, b1, W2, b2, W3, b3, W4, b4, W5, b5):
    src = edge_index[0].astype(jnp.int32)
    dst = edge_index[1].astype(jnp.int32)
    pad = jnp.full((E_PAD - src.shape[0],), N, jnp.int32)
    src1d = jnp.concatenate([src, pad])
    dst1d = jnp.concatenate([dst, pad])

    x_p = jnp.pad(x, ((0, N_PAD - N), (0, 0)))
    batch_p = jnp.pad(batch.astype(jnp.int32), (0, N_PAD - N))[:, None]

    w1p = jnp.pad(W1, ((0, 0), (0, 12)))
    b1p = jnp.pad(b1, (0, 12))[None, :]
    w2p = jnp.pad(W2, ((0, 12), (0, 8)))
    b2p = jnp.pad(b2, (0, 8))[None, :]
    w3p = jnp.pad(W3, ((0, 8), (0, 0)))
    b3p = b3[None, :]
    eye16 = jnp.eye(16, dtype=_f32)

    y_deg = jnp.zeros((N_PAD, 16), _f32).at[:N, 0].set(1.0)
    u_deg = _sc_scatter(y_deg, src1d, dst1d, 16, 128)

    y1, dinv = _tc_b1(x_p, w1p, u_deg)

    u1 = _sc_scatter(y1, src1d, dst1d, 16, 128)
    y2 = _tc_mid(u1, y1, dinv, eye16, b1p, 16)
    u2 = _sc_scatter(y2, src1d, dst1d, 16, 128)
    y3 = _tc_mid(u2, y2, dinv, w2p, b2p, 16)
    u3 = _sc_scatter(y3, src1d, dst1d, 16, 128)
    y4 = _tc_mid(u3, y3, dinv, w3p, b3p, 32)
    u4 = _sc_scatter(y4, src1d, dst1d, 32, 128)
    y5 = _tc_mid(u4, y4, dinv, W4, b4[None, :], 128)
    u5 = _sc_scatter(y5, src1d, dst1d, 128, 128)
    return _tc_final(u5, y5, dinv, W5, b5[None, :], batch_p)


# speedup vs baseline: 17.3628x; 1.5909x over previous
import functools

import jax
import jax.numpy as jnp
from jax import lax
from jax.experimental import pallas as pl
from jax.experimental.pallas import tpu as pltpu
from jax.experimental.pallas import tpu_sc as plsc

N = 10000
NUM_GRAPHS = 64
N_PAD = 10240
E_PAD = 327680
GROUPS = E_PAD // 128
GROUPS_PER_TILE = GROUPS // 32
ROWS_PER_SUBCORE = N_PAD // 16

_f32 = jnp.float32


_MESH = plsc.VectorSubcoreMesh(core_axis_name="c", subcore_axis_name="s")
_SC_PARAMS = pltpu.CompilerParams(use_tc_tiling_on_sc=False)


@functools.lru_cache(maxsize=None)
def _make_sc_scatter(d: int, ch: int, pipelined: bool = True):
    per_tile = E_PAD // 32
    assert per_tile % ch == 0
    steps = per_tile // ch

    if not pipelined:
        def body_simple(y_hbm, src_hbm, dst_hbm, zero_hbm, u_hbm,
                        src_v, dst_v, rows_v, u_sh, sem):
            c = lax.axis_index("c")
            s = lax.axis_index("s")
            r0 = s * ROWS_PER_SUBCORE
            pltpu.sync_copy(zero_hbm.at[pl.ds(r0, ROWS_PER_SUBCORE)],
                            u_sh.at[pl.ds(r0, ROWS_PER_SUBCORE)])
            tile = c * 16 + s
            pltpu.sync_copy(src_hbm.at[tile], src_v)
            pltpu.sync_copy(dst_hbm.at[tile], dst_v)
            plsc.subcore_barrier()

            def step(j, carry):
                pltpu.async_copy(y_hbm.at[src_v.at[j]], rows_v, sem).wait()
                pltpu.sync_copy(rows_v, u_sh.at[dst_v.at[j]], add=True)
                return carry

            lax.fori_loop(0, steps, step, 0)
            plsc.subcore_barrier()
            pltpu.sync_copy(u_sh.at[pl.ds(r0, ROWS_PER_SUBCORE)],
                            u_hbm.at[c, pl.ds(r0, ROWS_PER_SUBCORE)])

        return pl.kernel(
            body_simple,
            out_type=jax.ShapeDtypeStruct((2, N_PAD, d), _f32),
            mesh=_MESH,
            scratch_types=[
                pltpu.VMEM((steps, ch), jnp.int32),
                pltpu.VMEM((steps, ch), jnp.int32),
                pltpu.VMEM((ch, d), _f32),
                pltpu.VMEM_SHARED((N_PAD, d), _f32),
                pltpu.SemaphoreType.DMA,
            ],
            compiler_params=_SC_PARAMS,
        )

    def body(y_hbm, src_hbm, dst_hbm, zero_hbm, u_hbm,
             src_v, dst_v, rows_a, rows_b, u_sh, gsem_a, gsem_b):
        c = lax.axis_index("c")
        s = lax.axis_index("s")
        r0 = s * ROWS_PER_SUBCORE
        pltpu.sync_copy(zero_hbm.at[pl.ds(r0, ROWS_PER_SUBCORE)],
                        u_sh.at[pl.ds(r0, ROWS_PER_SUBCORE)])
        tile = c * 16 + s
        pltpu.sync_copy(src_hbm.at[tile], src_v)
        pltpu.sync_copy(dst_hbm.at[tile], dst_v)
        plsc.subcore_barrier()

        pltpu.async_copy(y_hbm.at[src_v.at[0]], rows_a, gsem_a)

        def step(j, carry):
            def run(rows, gsem, rows_n, gsem_n):
                pltpu.make_async_copy(y_hbm.at[src_v.at[j]], rows, gsem).wait()

                @pl.when(j + 1 < steps)
                def _():
                    pltpu.async_copy(y_hbm.at[src_v.at[j + 1]], rows_n, gsem_n)

                pltpu.sync_copy(rows, u_sh.at[dst_v.at[j]], add=True)

            @pl.when(j % 2 == 0)
            def _():
                run(rows_a, gsem_a, rows_b, gsem_b)

            @pl.when(j % 2 == 1)
            def _():
                run(rows_b, gsem_b, rows_a, gsem_a)

            return carry

        lax.fori_loop(0, steps, step, 0)
        plsc.subcore_barrier()
        pltpu.sync_copy(u_sh.at[pl.ds(r0, ROWS_PER_SUBCORE)],
                        u_hbm.at[c, pl.ds(r0, ROWS_PER_SUBCORE)])

    return pl.kernel(
        body,
        out_type=jax.ShapeDtypeStruct((2, N_PAD, d), _f32),
        mesh=_MESH,
        scratch_types=[
            pltpu.VMEM((steps, ch), jnp.int32),
            pltpu.VMEM((steps, ch), jnp.int32),
            pltpu.VMEM((ch, d), _f32),
            pltpu.VMEM((ch, d), _f32),
            pltpu.VMEM_SHARED((N_PAD, d), _f32),
            pltpu.SemaphoreType.DMA,
            pltpu.SemaphoreType.DMA,
        ],
        compiler_params=_SC_PARAMS,
    )


@functools.lru_cache(maxsize=None)
def _make_sc_scatter_split(dh: int, ch: int):
    per_tile = E_PAD // 16
    assert per_tile % ch == 0
    steps = per_tile // ch

    def body(y_hbm, src_hbm, dst_hbm, zero_hbm, u_hbm,
             src_v, dst_v, rows_a, rows_b, u_sh, gsem_a, gsem_b):
        c = lax.axis_index("c")
        s = lax.axis_index("s")
        r0 = s * ROWS_PER_SUBCORE
        pltpu.sync_copy(zero_hbm.at[pl.ds(r0, ROWS_PER_SUBCORE)],
                        u_sh.at[pl.ds(r0, ROWS_PER_SUBCORE)])
        pltpu.sync_copy(src_hbm.at[s], src_v)
        pltpu.sync_copy(dst_hbm.at[s], dst_v)
        plsc.subcore_barrier()
        yc = y_hbm.at[c]

        pltpu.async_copy(yc.at[src_v.at[0]], rows_a, gsem_a)

        def step(j, carry):
            def run(rows, gsem, rows_n, gsem_n):
                pltpu.make_async_copy(yc.at[src_v.at[j]], rows, gsem).wait()

                @pl.when(j + 1 < steps)
                def _():
                    pltpu.async_copy(yc.at[src_v.at[j + 1]], rows_n, gsem_n)

                pltpu.sync_copy(rows, u_sh.at[dst_v.at[j]], add=True)

            @pl.when(j % 2 == 0)
            def _():
                run(rows_a, gsem_a, rows_b, gsem_b)

            @pl.when(j % 2 == 1)
            def _():
                run(rows_b, gsem_b, rows_a, gsem_a)

            return carry

        lax.fori_loop(0, steps, step, 0)
        plsc.subcore_barrier()
        pltpu.sync_copy(u_sh.at[pl.ds(r0, ROWS_PER_SUBCORE)],
                        u_hbm.at[c, pl.ds(r0, ROWS_PER_SUBCORE)])

    return pl.kernel(
        body,
        out_type=jax.ShapeDtypeStruct((2, N_PAD, dh), _f32),
        mesh=_MESH,
        scratch_types=[
            pltpu.VMEM((steps, ch), jnp.int32),
            pltpu.VMEM((steps, ch), jnp.int32),
            pltpu.VMEM((ch, dh), _f32),
            pltpu.VMEM((ch, dh), _f32),
            pltpu.VMEM_SHARED((N_PAD, dh), _f32),
            pltpu.SemaphoreType.DMA,
            pltpu.SemaphoreType.DMA,
        ],
        compiler_params=_SC_PARAMS,
    )


def _sc_scatter_split(y2h, src1d, dst1d, dh, ch):
    per_tile = E_PAD // 16
    steps = per_tile // ch
    zero = jnp.zeros((N_PAD, dh), _f32)
    src3 = src1d.reshape(16, steps, ch)
    dst3 = dst1d.reshape(16, steps, ch)
    return _make_sc_scatter_split(dh, ch)(y2h, src3, dst3, zero)


def _sc_scatter(y, src1d, dst1d, d, ch, pipelined=True):
    per_tile = E_PAD // 32
    steps = per_tile // ch
    zero = jnp.zeros((N_PAD, d), _f32)
    src3 = src1d.reshape(32, steps, ch)
    dst3 = dst1d.reshape(32, steps, ch)
    return _make_sc_scatter(d, ch, pipelined)(y, src3, dst3, zero)


@functools.lru_cache(maxsize=None)
def _make_sc_degree(ch: int):
    per_tile = E_PAD // 32
    steps = per_tile // ch

    def body(dst_hbm, zero_hbm, ones_hbm, u_hbm, dst_v, ones_v, u_sh):
        c = lax.axis_index("c")
        s = lax.axis_index("s")
        r0 = s * ROWS_PER_SUBCORE
        pltpu.sync_copy(zero_hbm.at[pl.ds(r0, ROWS_PER_SUBCORE)],
                        u_sh.at[pl.ds(r0, ROWS_PER_SUBCORE)])
        pltpu.sync_copy(dst_hbm.at[c * 16 + s], dst_v)
        pltpu.sync_copy(ones_hbm, ones_v)
        plsc.subcore_barrier()

        def step(j, carry):
            pltpu.sync_copy(ones_v, u_sh.at[dst_v.at[j]], add=True)
            return carry

        lax.fori_loop(0, steps, step, 0)
        plsc.subcore_barrier()
        pltpu.sync_copy(u_sh.at[pl.ds(r0, ROWS_PER_SUBCORE)],
                        u_hbm.at[c, pl.ds(r0, ROWS_PER_SUBCORE)])

    return pl.kernel(
        body,
        out_type=jax.ShapeDtypeStruct((2, N_PAD, 16), _f32),
        mesh=_MESH,
        scratch_types=[
            pltpu.VMEM((steps, ch), jnp.int32),
            pltpu.VMEM((ch, 16), _f32),
            pltpu.VMEM_SHARED((N_PAD, 16), _f32),
        ],
        compiler_params=_SC_PARAMS,
    )


def _sc_degree(dst1d, ch):
    per_tile = E_PAD // 32
    steps = per_tile // ch
    zero = jnp.zeros((N_PAD, 16), _f32)
    ones = jnp.zeros((ch, 16), _f32).at[:, 0].set(1.0)
    dst3 = dst1d.reshape(32, steps, ch)
    return _make_sc_degree(ch)(dst3, zero, ones)


def _rowmask():
    rows = lax.broadcasted_iota(jnp.int32, (N_PAD, 1), 0)
    return (rows < N).astype(_f32)


def _tc_b1(x_p, w1p, u_deg):
    def body(x_ref, w_ref, ud_ref, y_ref, dinv_ref):
        deg = ud_ref[0, :, :1] + ud_ref[1, :, :1] + 1.0
        dinv = lax.rsqrt(deg)
        t = jnp.dot(x_ref[...], w_ref[...], preferred_element_type=_f32)
        y_ref[...] = _rowmask() * (dinv * t)
        dinv_ref[...] = dinv

    return pl.pallas_call(
        body,
        out_shape=(jax.ShapeDtypeStruct((N_PAD, 4), _f32),
                   jax.ShapeDtypeStruct((N_PAD, 1), _f32)),
    )(x_p, w1p, u_deg)


def _tc_mid(u, y, dinv, w, b, dout, split=False):
    def body(u_ref, y_ref, dinv_ref, w_ref, b_ref, out_ref):
        dinv = dinv_ref[...]
        g = dinv * (u_ref[0] + u_ref[1] + y_ref[...])
        h = jnp.maximum(jnp.dot(g, w_ref[...], preferred_element_type=_f32)
                        + b_ref[...], 0.0)
        yn = _rowmask() * (dinv * h)
        if split:
            out_ref[0] = yn[:, :dout // 2]
            out_ref[1] = yn[:, dout // 2:]
        else:
            out_ref[...] = yn

    out_shape = (jax.ShapeDtypeStruct((2, N_PAD, dout // 2), _f32) if split
                 else jax.ShapeDtypeStruct((N_PAD, dout), _f32))
    return pl.pallas_call(body, out_shape=out_shape)(u, y, dinv, w, b)


def _tc_final(u, y, dinv, w, b, batch_p):
    def body(u_ref, y_ref, dinv_ref, w_ref, b_ref, bat_ref, out_ref):
        gh = dinv_ref[...] * (u_ref[...] + y_ref[...])
        g = jnp.concatenate([gh[0], gh[1]], axis=1)
        h = jnp.maximum(jnp.dot(g, w_ref[...], preferred_element_type=_f32)
                        + b_ref[...], 0.0)
        gid = lax.broadcasted_iota(jnp.int32, (N_PAD, NUM_GRAPHS), 1)
        rows = lax.broadcasted_iota(jnp.int32, (N_PAD, NUM_GRAPHS), 0)
        m = ((bat_ref[...] == gid) & (rows < N)).astype(_f32)
        sums = lax.dot_general(m, h, (((0,), (0,)), ((), ())),
                               preferred_element_type=_f32)
        counts = jnp.sum(m, axis=0)[:, None]
        out_ref[...] = sums / jnp.maximum(counts, 1.0)

    return pl.pallas_call(
        body,
        out_shape=jax.ShapeDtypeStruct((NUM_GRAPHS, 256), _f32),
    )(u, y, dinv, w, b, batch_p)


def kernel(x, edge_index, batch, W1, b1, W2, b2, W3, b3, W4, b4, W5, b5):
    src = edge_index[0].astype(jnp.int32)
    dst = edge_index[1].astype(jnp.int32)
    pad = jnp.full((E_PAD - src.shape[0],), N, jnp.int32)
    src1d = jnp.concatenate([src, pad])
    dst1d = jnp.concatenate([dst, pad])

    x_p = jnp.pad(x, ((0, N_PAD - N), (0, 0)))
    batch_p = jnp.pad(batch.astype(jnp.int32), (0, N_PAD - N))[:, None]

    eye4 = jnp.eye(4, dtype=_f32)

    y_deg = jnp.zeros((N_PAD, 4), _f32).at[:N, 0].set(1.0)
    u_deg = _sc_scatter(y_deg, src1d, dst1d, 4, 128)

    y1, dinv = _tc_b1(x_p, W1, u_deg)

    u1 = _sc_scatter(y1, src1d, dst1d, 4, 128)
    y2 = _tc_mid(u1, y1, dinv, eye4, b1[None, :], 4)
    u2 = _sc_scatter(y2, src1d, dst1d, 4, 128)
    y3 = _tc_mid(u2, y2, dinv, W2, b2[None, :], 8)
    u3 = _sc_scatter(y3, src1d, dst1d, 8, 128)
    y4 = _tc_mid(u3, y3, dinv, W3, b3[None, :], 32)
    u4 = _sc_scatter(y4, src1d, dst1d, 32, 128)
    y5 = _tc_mid(u4, y4, dinv, W4, b4[None, :], 128, split=True)
    u5 = _sc_scatter_split(y5, src1d, dst1d, 64, 128)
    return _tc_final(u5, y5, dinv, W5, b5[None, :], batch_p)


# speedup vs baseline: 20.1189x; 1.1587x over previous
import functools

import jax
import jax.numpy as jnp
from jax import lax
from jax.experimental import pallas as pl
from jax.experimental.pallas import tpu as pltpu
from jax.experimental.pallas import tpu_sc as plsc

N = 10000
NUM_GRAPHS = 64
N_PAD = 10112
E_PAD = 327680
GROUPS = E_PAD // 128
GROUPS_PER_TILE = GROUPS // 32
ROWS_PER_SUBCORE = N_PAD // 16

_f32 = jnp.float32


_MESH = plsc.VectorSubcoreMesh(core_axis_name="c", subcore_axis_name="s")
_SC_PARAMS = pltpu.CompilerParams(use_tc_tiling_on_sc=False)


@functools.lru_cache(maxsize=None)
def _make_sc_pipe(d: int, ch: int, split: bool):
    n_tiles = 16 if split else 32
    per_tile = E_PAD // n_tiles
    assert per_tile % (4 * ch) == 0
    steps = per_tile // ch
    groups = steps // 4

    def body(y_hbm, src_hbm, dst_hbm, zero_hbm, u_hbm,
             src_v, dst_v, *scratch):
        rows = scratch[:8]
        u_sh = scratch[8]
        gsem = scratch[9:17]
        ssem = scratch[17:]
        c = lax.axis_index("c")
        s = lax.axis_index("s")
        r0 = s * ROWS_PER_SUBCORE
        pltpu.sync_copy(zero_hbm.at[pl.ds(r0, ROWS_PER_SUBCORE)],
                        u_sh.at[pl.ds(r0, ROWS_PER_SUBCORE)])
        tile = s if split else c * 16 + s
        pltpu.sync_copy(src_hbm.at[tile], src_v)
        pltpu.sync_copy(dst_hbm.at[tile], dst_v)
        plsc.subcore_barrier()
        yc = y_hbm.at[c] if split else y_hbm

        def gather(j, k, sem):
            return pltpu.async_copy(yc.at[src_v.at[j]], rows[k], sem)

        def gather_wait(j, k, sem):
            pltpu.make_async_copy(yc.at[src_v.at[j]], rows[k], sem).wait()

        def scatter(j, k, sem):
            return pltpu.async_copy(rows[k], u_sh.at[dst_v.at[j]], sem,
                                    add=True)

        def scatter_wait(j, k, sem):
            pltpu.make_async_copy(rows[k], u_sh.at[dst_v.at[j]],
                                  sem).wait()

        for i in range(4):
            gather(i, i, gsem[i])

        def grp(g, parity):
            p = parity * 4
            q = (1 - parity) * 4
            for i in range(4):
                j = 4 * g + i
                gather_wait(j, p + i, gsem[p + i])
                scatter(j, p + i, ssem[p + i])
            for i in range(4):
                @pl.when(g > 0)
                def _():
                    scatter_wait(4 * (g - 1) + i, q + i, ssem[q + i])

                @pl.when(g + 1 < groups)
                def _():
                    gather(4 * (g + 1) + i, q + i, gsem[q + i])

        def step(g, carry):
            @pl.when(g % 2 == 0)
            def _():
                grp(g, 0)

            @pl.when(g % 2 == 1)
            def _():
                grp(g, 1)

            return carry

        lax.fori_loop(0, groups, step, 0)
        lastp = ((groups - 1) % 2) * 4
        for i in range(4):
            scatter_wait(4 * (groups - 1) + i, lastp + i, ssem[lastp + i])
        plsc.subcore_barrier()
        pltpu.sync_copy(u_sh.at[pl.ds(r0, ROWS_PER_SUBCORE)],
                        u_hbm.at[c, pl.ds(r0, ROWS_PER_SUBCORE)])

    return pl.kernel(
        body,
        out_type=jax.ShapeDtypeStruct((2, N_PAD, d), _f32),
        mesh=_MESH,
        scratch_types=(
            [pltpu.VMEM((steps, ch), jnp.int32),
             pltpu.VMEM((steps, ch), jnp.int32)]
            + [pltpu.VMEM((ch, d), _f32)] * 8
            + [pltpu.VMEM_SHARED((N_PAD, d), _f32)]
            + [pltpu.SemaphoreType.DMA] * 16),
        compiler_params=_SC_PARAMS,
    )


def _sc_scatter(y, src1d, dst1d, d, ch):
    per_tile = E_PAD // 32
    steps = per_tile // ch
    zero = jnp.zeros((N_PAD, d), _f32)
    src3 = src1d.reshape(32, steps, ch)
    dst3 = dst1d.reshape(32, steps, ch)
    return _make_sc_pipe(d, ch, False)(y, src3, dst3, zero)


def _sc_scatter_split(y2h, src1d, dst1d, dh, ch):
    per_tile = E_PAD // 16
    steps = per_tile // ch
    zero = jnp.zeros((N_PAD, dh), _f32)
    src3 = src1d.reshape(16, steps, ch)
    dst3 = dst1d.reshape(16, steps, ch)
    return _make_sc_pipe(dh, ch, True)(y2h, src3, dst3, zero)


def _rowmask():
    rows = lax.broadcasted_iota(jnp.int32, (N_PAD, 1), 0)
    return (rows < N).astype(_f32)


def _tc_b1(x_p, w1p, u_deg):
    def body(x_ref, w_ref, ud_ref, y_ref, dinv_ref):
        deg = ud_ref[0, :, :1] + ud_ref[1, :, :1] + 1.0
        dinv = lax.rsqrt(deg)
        t = jnp.dot(x_ref[...], w_ref[...], preferred_element_type=_f32)
        y_ref[...] = _rowmask() * (dinv * t)
        dinv_ref[...] = dinv

    return pl.pallas_call(
        body,
        out_shape=(jax.ShapeDtypeStruct((N_PAD, 8), _f32),
                   jax.ShapeDtypeStruct((N_PAD, 1), _f32)),
    )(x_p, w1p, u_deg)


def _tc_mid(u, y, dinv, w, b, dout, split=False, in_split=False, nsplit=2):
    def body(u_ref, y_ref, dinv_ref, w_ref, b_ref, out_ref):
        dinv = dinv_ref[...]
        if in_split:
            ns = u_ref.shape[0]
            ws = w_ref.shape[0] // ns
            acc = b_ref[...]
            for k in range(ns):
                gk = dinv * (u_ref[k] + y_ref[k])
                acc = acc + jnp.dot(gk, w_ref[k * ws:(k + 1) * ws, :],
                                    preferred_element_type=_f32)
            h = jnp.maximum(acc, 0.0)
        else:
            g = dinv * (u_ref[0] + u_ref[1] + y_ref[...])
            h = jnp.maximum(
                jnp.dot(g, w_ref[...], preferred_element_type=_f32)
                + b_ref[...], 0.0)
        yn = _rowmask() * (dinv * h)
        if split:
            w_ = dout // nsplit
            for k in range(nsplit):
                out_ref[k] = yn[:, k * w_:(k + 1) * w_]
        else:
            out_ref[...] = yn

    out_shape = (jax.ShapeDtypeStruct((nsplit, N_PAD, dout // nsplit), _f32)
                 if split else jax.ShapeDtypeStruct((N_PAD, dout), _f32))
    return pl.pallas_call(body, out_shape=out_shape)(u, y, dinv, w, b)


def _tc_final(u, y, dinv, w, b, batch_p):
    nblk = 4
    blk = N_PAD // nblk
    ns = u.shape[0]
    ds_ = u.shape[2]

    def body(u_ref, y_ref, dinv_ref, w_ref, b_ref, bat_ref, out_ref, cnt_ref):
        i = pl.program_id(0)

        @pl.when(i == 0)
        def _():
            out_ref[...] = jnp.zeros_like(out_ref)
            cnt_ref[...] = jnp.zeros_like(cnt_ref)

        ws = w_ref.shape[0] // ns
        acc = b_ref[...]
        for k in range(ns):
            gk = dinv_ref[...] * (u_ref[k] + y_ref[k])
            acc = acc + jnp.dot(gk, w_ref[k * ws:(k + 1) * ws, :],
                                preferred_element_type=_f32)
        h = jnp.maximum(acc, 0.0)
        gid = lax.broadcasted_iota(jnp.int32, (blk, NUM_GRAPHS), 1)
        rows = i * blk + lax.broadcasted_iota(jnp.int32, (blk, NUM_GRAPHS), 0)
        m = ((bat_ref[...] == gid) & (rows < N)).astype(_f32)
        out_ref[...] += lax.dot_general(m, h, (((0,), (0,)), ((), ())),
                                        preferred_element_type=_f32)
        cnt_ref[...] += jnp.sum(m, axis=0)[:, None]

        @pl.when(i == nblk - 1)
        def _():
            out_ref[...] = out_ref[...] / jnp.maximum(cnt_ref[...], 1.0)

    return pl.pallas_call(
        body,
        grid=(nblk,),
        in_specs=[
            pl.BlockSpec((ns, blk, ds_), lambda i: (0, i, 0)),
            pl.BlockSpec((ns, blk, ds_), lambda i: (0, i, 0)),
            pl.BlockSpec((blk, 1), lambda i: (i, 0)),
            pl.BlockSpec(w.shape, lambda i: (0, 0)),
            pl.BlockSpec(b.shape, lambda i: (0, 0)),
            pl.BlockSpec((blk, 1), lambda i: (i, 0)),
        ],
        out_specs=pl.BlockSpec((NUM_GRAPHS, 256), lambda i: (0, 0)),
        out_shape=jax.ShapeDtypeStruct((NUM_GRAPHS, 256), _f32),
        scratch_shapes=[pltpu.VMEM((NUM_GRAPHS, 1), _f32)],
    )(u, y, dinv, w, b, batch_p)


def kernel(x, edge_index, batch, W1, b1, W2, b2, W3, b3, W4, b4, W5, b5):
    src = edge_index[0].astype(jnp.int32)
    dst = edge_index[1].astype(jnp.int32)
    pad = jnp.full((E_PAD - src.shape[0],), N, jnp.int32)
    src1d = jnp.concatenate([src, pad])
    dst1d = jnp.concatenate([dst, pad])

    x_p = jnp.pad(x, ((0, N_PAD - N), (0, 0)))
    batch_p = jnp.pad(batch.astype(jnp.int32), (0, N_PAD - N))[:, None]

    w1p = jnp.pad(W1, ((0, 0), (0, 4)))
    b1p = jnp.pad(b1, (0, 4))[None, :]
    w2p = jnp.pad(W2, ((0, 4), (0, 0)))
    eye8 = jnp.eye(8, dtype=_f32)

    y_deg = jnp.zeros((N_PAD, 8), _f32).at[:N, 0].set(1.0)
    u_deg = _sc_scatter(y_deg, src1d, dst1d, 8, 128)

    y1, dinv = _tc_b1(x_p, w1p, u_deg)

    u1 = _sc_scatter(y1, src1d, dst1d, 8, 128)
    y2 = _tc_mid(u1, y1, dinv, eye8, b1p, 8)
    u2 = _sc_scatter(y2, src1d, dst1d, 8, 128)
    y3 = _tc_mid(u2, y2, dinv, w2p, b2[None, :], 8)
    u3 = _sc_scatter(y3, src1d, dst1d, 8, 128)
    y4 = _tc_mid(u3, y3, dinv, W3, b3[None, :], 32, split=True)
    u4 = _sc_scatter_split(y4, src1d, dst1d, 16, 128)
    y5 = _tc_mid(u4, y4, dinv, W4, b4[None, :], 128,
                 split=True, in_split=True, nsplit=4)
    u5a = _sc_scatter_split(y5[:2], src1d, dst1d, 32, 128)
    u5b = _sc_scatter_split(y5[2:], src1d, dst1d, 32, 128)
    u5 = jnp.concatenate([u5a, u5b], axis=0)
    return _tc_final(u5, y5, dinv, W5, b5[None, :], batch_p)
